# baseline (speedup 1.0000x reference)
import functools

import jax
import jax.numpy as jnp
from jax import lax
from jax.experimental import pallas as pl
from jax.experimental.pallas import tpu as pltpu
from jax.experimental.pallas import tpu_sc as plsc

N_NODES = 1000000
N_RELATIONS = 1000
EMBED_DIM = 32
BATCH = 16384

_INFO = plsc.get_sparse_core_info()
_NC = _INFO.num_cores
_NS = _INFO.num_subcores
_NW = _NC * _NS
_L = _INFO.num_lanes

_B_PER_W = BATCH // _NW
_CHUNK = 256
_N_CHUNKS = _B_PER_W // _CHUNK
_GROUPS = _CHUNK // _L


def _body(heads_hbm, rels_hbm, tails_hbm,
          nTr_hbm, nTi_hbm, rTr_hbm, rTi_hbm,
          out_hbm,
          h_idx, r_idx, t_idx,
          srT, siT, trT, tiT,
          relr_v, reli_v,
          out_v, sem, rsem):
    wid = lax.axis_index("s") * _NC + lax.axis_index("c")
    base = wid * _B_PER_W

    rel_copies = []
    for c in range(EMBED_DIM):
        rel_copies.append(pltpu.async_copy(
            rTr_hbm.at[c], relr_v.at[pl.ds(c * N_RELATIONS, N_RELATIONS)],
            rsem))
        rel_copies.append(pltpu.async_copy(
            rTi_hbm.at[c], reli_v.at[pl.ds(c * N_RELATIONS, N_RELATIONS)],
            rsem))

    pltpu.sync_copy(heads_hbm.at[pl.ds(base, _B_PER_W)], h_idx)
    pltpu.sync_copy(rels_hbm.at[pl.ds(base, _B_PER_W)], r_idx)
    pltpu.sync_copy(tails_hbm.at[pl.ds(base, _B_PER_W)], t_idx)

    for cp in rel_copies:
        cp.wait()

    for chunk in range(_N_CHUNKS):
        off = chunk * _CHUNK
        h_ids = h_idx.at[pl.ds(off, _CHUNK)]
        t_ids = t_idx.at[pl.ds(off, _CHUNK)]
        copies = []
        for c in range(EMBED_DIM):
            copies.append(
                pltpu.async_copy(nTr_hbm.at[c].at[h_ids], srT.at[c], sem))
            copies.append(
                pltpu.async_copy(nTi_hbm.at[c].at[h_ids], siT.at[c], sem))
            copies.append(
                pltpu.async_copy(nTr_hbm.at[c].at[t_ids], trT.at[c], sem))
            copies.append(
                pltpu.async_copy(nTi_hbm.at[c].at[t_ids], tiT.at[c], sem))
        for cp in copies:
            cp.wait()

        for g in range(_GROUPS):
            goff = g * _L
            rel_ids = r_idx[pl.ds(off + goff, _L)]
            phi = jnp.zeros((_L,), jnp.float32)
            for c in range(EMBED_DIM):
                flat_ids = rel_ids + (c * N_RELATIONS)
                sr_c = srT[c, pl.ds(goff, _L)]
                si_c = siT[c, pl.ds(goff, _L)]
                tr_c = trT[c, pl.ds(goff, _L)]
                ti_c = tiT[c, pl.ds(goff, _L)]
                rr_c = plsc.load_gather(relr_v, [flat_ids])
                ri_c = plsc.load_gather(reli_v, [flat_ids])
                phi = phi + rr_c * (sr_c * tr_c + si_c * ti_c)
                phi = phi + ri_c * (sr_c * ti_c - si_c * tr_c)
            out_v[pl.ds(off + goff, _L)] = phi

    pltpu.sync_copy(out_v, out_hbm.at[pl.ds(base, _B_PER_W)])


@jax.jit
def kernel(heads, rels, tails, node_r, node_i, rel_r, rel_i):
    mesh = plsc.VectorSubcoreMesh(core_axis_name="c", subcore_axis_name="s")
    f = functools.partial(
        pl.kernel,
        out_type=jax.ShapeDtypeStruct((BATCH,), jnp.float32),
        mesh=mesh,
        compiler_params=pltpu.CompilerParams(
            use_tc_tiling_on_sc=False, needs_layout_passes=False),
        scratch_types=[
            pltpu.VMEM((_B_PER_W,), jnp.int32),
            pltpu.VMEM((_B_PER_W,), jnp.int32),
            pltpu.VMEM((_B_PER_W,), jnp.int32),
            pltpu.VMEM((EMBED_DIM, _CHUNK), jnp.float32),
            pltpu.VMEM((EMBED_DIM, _CHUNK), jnp.float32),
            pltpu.VMEM((EMBED_DIM, _CHUNK), jnp.float32),
            pltpu.VMEM((EMBED_DIM, _CHUNK), jnp.float32),
            pltpu.VMEM((EMBED_DIM * N_RELATIONS,), jnp.float32),
            pltpu.VMEM((EMBED_DIM * N_RELATIONS,), jnp.float32),
            pltpu.VMEM((_B_PER_W,), jnp.float32),
            pltpu.SemaphoreType.DMA,
            pltpu.SemaphoreType.DMA,
        ],
    )(_body)
    return f(heads, rels, tails, node_r.T, node_i.T, rel_r.T, rel_i.T)
